# baseline (speedup 1.0000x reference)
import functools

import jax
import jax.numpy as jnp
from jax import lax
from jax.experimental import pallas as pl
from jax.experimental.pallas import tpu as pltpu
from jax.experimental.pallas import tpu_sc as plsc

_B = 16384
_D = 64
_TOT = 2 * _B
_CH = 128
_NW = 32
_RPW = _TOT // _NW
_HPW = _RPW // 2
_CPW = _RPW // _CH
_L = 16
_GPC = _CH // _L


def _sc_sumsq(ent, rel, h1, r1, t1):
    mesh = plsc.VectorSubcoreMesh(core_axis_name="c", subcore_axis_name="s")

    @functools.partial(
        pl.kernel,
        mesh=mesh,
        out_type=jax.ShapeDtypeStruct((_TOT,), jnp.float32),
        compiler_params=pltpu.CompilerParams(needs_layout_passes=False),
        scratch_types=[
            pltpu.VMEM((_RPW,), jnp.int32),
            pltpu.VMEM((_RPW,), jnp.int32),
            pltpu.VMEM((_RPW,), jnp.int32),
            pltpu.VMEM((_CH, 2 * _D), jnp.float32),
            pltpu.VMEM((_CH, 2 * _D), jnp.float32),
            pltpu.VMEM((_CH, 2 * _D), jnp.float32),
            pltpu.VMEM((_CH, 2 * _D), jnp.float32),
            pltpu.VMEM((_CH, 2 * _D), jnp.float32),
            pltpu.VMEM((_CH, 2 * _D), jnp.float32),
            pltpu.VMEM((64 * 128,), jnp.int32),
            pltpu.VMEM((_RPW,), jnp.float32),
            pltpu.SemaphoreType.DMA,
            pltpu.SemaphoreType.DMA,
        ],
    )
    def k(ent_hbm, rel_hbm, h_hbm, r_hbm, t_hbm, out_hbm,
          hi, ri, ti, ha, ra, ta, hb, rb, tb, dmy, ss, semA, semB):
        wid = lax.axis_index("s") * 2 + lax.axis_index("c")
        rb0 = wid * _RPW
        pltpu.sync_copy(h_hbm.at[pl.ds(rb0, _RPW)], hi)
        pltpu.sync_copy(r_hbm.at[pl.ds(rb0, _RPW)], ri)
        pltpu.sync_copy(t_hbm.at[pl.ds(rb0, _RPW)], ti)

        lanes = lax.iota(jnp.int32, _L)

        def fire(kk, bh, br, bt, sem):
            def grp(g, carry):
                off = kk * _CH + g * _L
                hv = hi[pl.ds(off, _L)]
                rv = ri[pl.ds(off, _L)]
                tv = ti[pl.ds(off, _L)]
                for j in range(_L):
                    row = g * _L + j
                    pltpu.async_copy(
                        ent_hbm.at[hv[j]], bh.at[row, pl.ds(0, _D)], sem)
                    pltpu.async_copy(
                        rel_hbm.at[rv[j]], br.at[row, pl.ds(0, _D)], sem)
                    pltpu.async_copy(
                        ent_hbm.at[tv[j]], bt.at[row, pl.ds(0, _D)], sem)
                return carry

            lax.fori_loop(0, _GPC, grp, 0)

        def drain(sem):
            for _ in range(3):
                pltpu.make_async_copy(h_hbm.at[pl.ds(0, 64 * 128)],
                                      dmy, sem).wait()

        def compute(kk, bh, br, bt):
            def grp(g, carry):
                off = kk * _CH + g * _L
                rows = lanes + g * _L
                acc = jnp.zeros((_L,), jnp.float32)
                for dd in range(_D):
                    dsp = jnp.full((_L,), dd, jnp.int32)
                    hv = plsc.load_gather(bh, [rows, dsp])
                    rv = plsc.load_gather(br, [rows, dsp])
                    tv = plsc.load_gather(bt, [rows, dsp])
                    e = hv + rv - tv
                    acc = acc + e * e
                ss[pl.ds(off, _L)] = acc
                return carry

            lax.fori_loop(0, _GPC, grp, 0)

        fire(0, ha, ra, ta, semA)

        def chunk_pair(c, carry):
            k0 = c * 2
            fire(k0 + 1, hb, rb, tb, semB)
            drain(semA)
            compute(k0, ha, ra, ta)

            @pl.when(c < (_CPW // 2 - 1))
            def _():
                fire(k0 + 2, ha, ra, ta, semA)

            drain(semB)
            compute(k0 + 1, hb, rb, tb)
            return carry

        lax.fori_loop(0, _CPW // 2, chunk_pair, 0)
        pltpu.sync_copy(ss, out_hbm.at[pl.ds(wid * _RPW, _RPW)])

    return k(ent, rel, h1, r1, t1)


def kernel(posX, negX, entityEmbedding, relationEmbedding, PP, AP,
           alpha, beta, sigma, lambda1, lambda2, lambda3):
    def interleave(a, b):
        return jnp.concatenate(
            [a.reshape(_NW, _HPW), b.reshape(_NW, _HPW)], axis=1).reshape(-1)

    h1 = interleave(posX[:, 0], negX[:, 0])
    r1 = interleave(posX[:, 1], negX[:, 1])
    t1 = interleave(posX[:, 2], negX[:, 2])

    ss = _sc_sumsq(entityEmbedding, relationEmbedding, h1, r1, t1)
    ssr = ss.reshape(_NW, 2, _HPW)
    ssp = ssr[:, 0, :].reshape(128, 128)
    ssn = ssr[:, 1, :].reshape(128, 128)

    prm = jnp.stack([alpha, 1.0 + beta, lambda1, lambda2, lambda3]).astype(
        jnp.float32)

    def body(par_ref, sp_ref, sn_ref, pp_ref, ap_ref, o_ref):
        a = par_ref[0]
        b1 = par_ref[1]
        l1 = par_ref[2]
        l2 = par_ref[3]
        l3 = par_ref[4]
        pos = jnp.sqrt(sp_ref[...] + 1e-12)
        neg = jnp.sqrt(sn_ref[...] + 1e-12)
        d = pos - neg + 1.0
        lt = jnp.where(d < 0, b1, a)
        cw = l1 * lt + l2 * pp_ref[...] + l3 * (1.0 / (1.0 + jnp.exp(-ap_ref[...])))
        o_ref[0, 0] = jnp.sum(jnp.maximum(d * cw, 0.0)) * (1.0 / _B)

    out = pl.pallas_call(
        body,
        out_shape=jax.ShapeDtypeStruct((1, 1), jnp.float32),
        in_specs=[pl.BlockSpec(memory_space=pltpu.SMEM)] +
                 [pl.BlockSpec(memory_space=pltpu.VMEM)] * 4,
        out_specs=pl.BlockSpec(memory_space=pltpu.SMEM),
    )(prm, ssp, ssn, PP.reshape(128, 128), AP.reshape(128, 128))
    probe_rows = jnp.take(entityEmbedding, posX[:, 0], axis=0)
    return out[0, 0] + 0.0 * probe_rows[0, 0]


# speedup vs baseline: 1.0237x; 1.0237x over previous
import functools

import jax
import jax.numpy as jnp
from jax import lax
from jax.experimental import pallas as pl
from jax.experimental.pallas import tpu as pltpu
from jax.experimental.pallas import tpu_sc as plsc

_B = 16384
_D = 64
_TOT = 2 * _B
_CH = 64
_NW = 32
_RPW = _TOT // _NW
_HPW = _RPW // 2
_CPW = _RPW // _CH
_L = 16
_GPC = _CH // _L


def _sc_sumsq(ent, rel, h1, r1, t1):
    mesh = plsc.VectorSubcoreMesh(core_axis_name="c", subcore_axis_name="s")

    @functools.partial(
        pl.kernel,
        mesh=mesh,
        out_type=jax.ShapeDtypeStruct((_TOT,), jnp.float32),
        compiler_params=pltpu.CompilerParams(needs_layout_passes=False),
        scratch_types=[
            pltpu.VMEM((_RPW,), jnp.int32),
            pltpu.VMEM((_RPW,), jnp.int32),
            pltpu.VMEM((_RPW,), jnp.int32),
        ] + [pltpu.VMEM((_CH, 2 * _D), jnp.float32)] * 12 + [
            pltpu.VMEM((4096,), jnp.int32),
            pltpu.VMEM((_RPW,), jnp.float32),
            pltpu.SemaphoreType.DMA,
            pltpu.SemaphoreType.DMA,
            pltpu.SemaphoreType.DMA,
            pltpu.SemaphoreType.DMA,
        ],
    )
    def k(ent_hbm, rel_hbm, h_hbm, r_hbm, t_hbm, out_hbm,
          hi, ri, ti,
          ha0, ra0, ta0, ha1, ra1, ta1, ha2, ra2, ta2, ha3, ra3, ta3,
          dmy, ss, sem0, sem1, sem2, sem3):
        wid = lax.axis_index("s") * 2 + lax.axis_index("c")
        rb0 = wid * _RPW
        pltpu.sync_copy(h_hbm.at[pl.ds(rb0, _RPW)], hi)
        pltpu.sync_copy(r_hbm.at[pl.ds(rb0, _RPW)], ri)
        pltpu.sync_copy(t_hbm.at[pl.ds(rb0, _RPW)], ti)

        lanes = lax.iota(jnp.int32, _L)

        def fire(kk, bh, br, bt, sem):
            def grp(g, carry):
                off = kk * _CH + g * _L
                hv = hi[pl.ds(off, _L)]
                rv = ri[pl.ds(off, _L)]
                tv = ti[pl.ds(off, _L)]
                for j in range(_L):
                    row = g * _L + j
                    pltpu.async_copy(
                        ent_hbm.at[hv[j]], bh.at[row, pl.ds(0, _D)], sem)
                    pltpu.async_copy(
                        rel_hbm.at[rv[j]], br.at[row, pl.ds(0, _D)], sem)
                    pltpu.async_copy(
                        ent_hbm.at[tv[j]], bt.at[row, pl.ds(0, _D)], sem)
                return carry

            lax.fori_loop(0, _GPC, grp, 0)

        def drain(sem):
            for _ in range(3):
                pltpu.make_async_copy(h_hbm.at[pl.ds(0, 4096)],
                                      dmy, sem).wait()

        def compute(kk, bh, br, bt):
            def grp(g, carry):
                off = kk * _CH + g * _L
                rows = lanes + g * _L
                acc = jnp.zeros((_L,), jnp.float32)
                for dd in range(_D):
                    dsp = jnp.full((_L,), dd, jnp.int32)
                    hv = plsc.load_gather(bh, [rows, dsp])
                    rv = plsc.load_gather(br, [rows, dsp])
                    tv = plsc.load_gather(bt, [rows, dsp])
                    e = hv + rv - tv
                    acc = acc + e * e
                ss[pl.ds(off, _L)] = acc
                return carry

            lax.fori_loop(0, _GPC, grp, 0)

        sets = ((ha0, ra0, ta0, sem0), (ha1, ra1, ta1, sem1),
                (ha2, ra2, ta2, sem2), (ha3, ra3, ta3, sem3))
        for s in range(3):
            fire(s, *sets[s])

        def chunk_quad(c, carry):
            k0 = c * 4
            fire(k0 + 3, *sets[3])
            for s in range(4):
                bh, br, bt, sem = sets[s]
                if s > 0:
                    @pl.when(c < (_CPW // 4 - 1))
                    def _(s=s, k0=k0):
                        fire(k0 + 3 + s, *sets[s - 1])
                drain(sem)
                compute(k0 + s, bh, br, bt)
            return carry

        lax.fori_loop(0, _CPW // 4, chunk_quad, 0)
        pltpu.sync_copy(ss, out_hbm.at[pl.ds(wid * _RPW, _RPW)])

    return k(ent, rel, h1, r1, t1)


def kernel(posX, negX, entityEmbedding, relationEmbedding, PP, AP,
           alpha, beta, sigma, lambda1, lambda2, lambda3):
    def interleave(a, b):
        return jnp.concatenate(
            [a.reshape(_NW, _HPW), b.reshape(_NW, _HPW)], axis=1).reshape(-1)

    h1 = interleave(posX[:, 0], negX[:, 0])
    r1 = interleave(posX[:, 1], negX[:, 1])
    t1 = interleave(posX[:, 2], negX[:, 2])

    ss = _sc_sumsq(entityEmbedding, relationEmbedding, h1, r1, t1)
    ssr = ss.reshape(_NW, 2, _HPW)
    ssp = ssr[:, 0, :].reshape(128, 128)
    ssn = ssr[:, 1, :].reshape(128, 128)

    prm = jnp.stack([alpha, 1.0 + beta, lambda1, lambda2, lambda3]).astype(
        jnp.float32)

    def body(par_ref, sp_ref, sn_ref, pp_ref, ap_ref, o_ref):
        a = par_ref[0]
        b1 = par_ref[1]
        l1 = par_ref[2]
        l2 = par_ref[3]
        l3 = par_ref[4]
        pos = jnp.sqrt(sp_ref[...] + 1e-12)
        neg = jnp.sqrt(sn_ref[...] + 1e-12)
        d = pos - neg + 1.0
        lt = jnp.where(d < 0, b1, a)
        cw = l1 * lt + l2 * pp_ref[...] + l3 * (1.0 / (1.0 + jnp.exp(-ap_ref[...])))
        o_ref[0, 0] = jnp.sum(jnp.maximum(d * cw, 0.0)) * (1.0 / _B)

    out = pl.pallas_call(
        body,
        out_shape=jax.ShapeDtypeStruct((1, 1), jnp.float32),
        in_specs=[pl.BlockSpec(memory_space=pltpu.SMEM)] +
                 [pl.BlockSpec(memory_space=pltpu.VMEM)] * 4,
        out_specs=pl.BlockSpec(memory_space=pltpu.SMEM),
    )(prm, ssp, ssn, PP.reshape(128, 128), AP.reshape(128, 128))
    return out[0, 0]
